# baseline (speedup 1.0000x reference)
import functools

import jax
import jax.numpy as jnp
from jax import lax
from jax.experimental import pallas as pl
from jax.experimental.pallas import tpu as pltpu
from jax.experimental.pallas import tpu_sc as plsc

EMBED_DIM = 64
VOCAB_PAD = 1024
NUM_CORES = 2
NUM_SUBCORES = 16
NUM_WORKERS = NUM_CORES * NUM_SUBCORES
T_PER_GROUP = 8
B_BLOCK = 128
LANES = 16


def _emb_kernel(ids_t_hbm, table_hbm, out_hbm,
                table_v, idx0, idx1, tr0, tr1,
                semi0, semi1, semw0, semw1):
    wid = lax.axis_index("s") * NUM_CORES + lax.axis_index("c")
    n_t, n_b = ids_t_hbm.shape
    n_tg = n_t // T_PER_GROUP
    n_bb = n_b // B_BLOCK
    groups = (n_tg * n_bb) // NUM_WORKERS
    g_base = wid * groups

    pltpu.sync_copy(table_hbm, table_v)

    def ids_start(k, idxv, sem):
        g_lin = g_base + k
        tg = g_lin // n_bb
        bb = g_lin % n_bb
        r0 = pl.multiple_of(tg * T_PER_GROUP, T_PER_GROUP)
        c0 = pl.multiple_of(bb * B_BLOCK, B_BLOCK)
        pltpu.async_copy(
            ids_t_hbm.at[pl.ds(r0, T_PER_GROUP), pl.ds(c0, B_BLOCK)],
            idxv, sem)

    def ids_wait(idxv, sem):
        pltpu.make_async_copy(
            ids_t_hbm.at[pl.ds(0, T_PER_GROUP), pl.ds(0, B_BLOCK)],
            idxv, sem).wait()

    def wait_write(tr, sem):
        pltpu.make_async_copy(tr, out_hbm.at[0, :, pl.ds(0, B_BLOCK)],
                              sem).wait()

    trs = (tr0, tr1)
    semws = (semw0, semw1)

    def do_group(k, idxv):
        g_lin = g_base + k
        tg = g_lin // n_bb
        bb = g_lin % n_bb
        c0 = pl.multiple_of(bb * B_BLOCK, B_BLOCK)
        for tt in range(T_PER_GROUP):
            tr = trs[tt % 2]
            semw = semws[tt % 2]

            @pl.when(k * T_PER_GROUP + tt >= 2)
            def _():
                wait_write(tr, semw)

            ids_g = [idxv[tt, pl.ds(g * LANES, LANES)]
                     for g in range(B_BLOCK // LANES)]

            @plsc.parallel_loop(0, EMBED_DIM, unroll=8)
            def dbody(d):
                base = d * VOCAB_PAD
                for g in range(B_BLOCK // LANES):
                    v = plsc.load_gather(table_v, [ids_g[g] + base])
                    tr[d, pl.ds(g * LANES, LANES)] = v
            t = tg * T_PER_GROUP + tt
            pltpu.async_copy(tr, out_hbm.at[t, :, pl.ds(c0, B_BLOCK)], semw)

    ids_start(0, idx0, semi0)
    ids_start(1, idx1, semi1)

    def outer(r, carry):
        k0 = 2 * r
        ids_wait(idx0, semi0)
        do_group(k0, idx0)

        @pl.when(k0 + 2 < groups)
        def _():
            ids_start(k0 + 2, idx0, semi0)
        ids_wait(idx1, semi1)
        do_group(k0 + 1, idx1)

        @pl.when(k0 + 3 < groups)
        def _():
            ids_start(k0 + 3, idx1, semi1)
        return carry

    lax.fori_loop(0, groups // 2, outer, 0)
    wait_write(tr0, semw0)
    wait_write(tr1, semw1)


def kernel(phoneme_ids, table):
    b, t = phoneme_ids.shape
    ids_t = jnp.transpose(phoneme_ids.astype(jnp.int32))
    table_t = jnp.zeros((EMBED_DIM, VOCAB_PAD), jnp.float32)
    table_t = table_t.at[:, :table.shape[0]].set(jnp.transpose(table))
    table_flat = table_t.reshape(-1)

    emb = functools.partial(
        pl.kernel,
        mesh=plsc.VectorSubcoreMesh(core_axis_name="c", subcore_axis_name="s"),
        out_type=jax.ShapeDtypeStruct((t, EMBED_DIM, b), jnp.float32),
        scratch_types=[
            pltpu.VMEM((EMBED_DIM * VOCAB_PAD,), jnp.float32),
            pltpu.VMEM((T_PER_GROUP, B_BLOCK), jnp.int32),
            pltpu.VMEM((T_PER_GROUP, B_BLOCK), jnp.int32),
            pltpu.VMEM((EMBED_DIM, B_BLOCK), jnp.float32),
            pltpu.VMEM((EMBED_DIM, B_BLOCK), jnp.float32),
            pltpu.SemaphoreType.DMA,
            pltpu.SemaphoreType.DMA,
            pltpu.SemaphoreType.DMA,
            pltpu.SemaphoreType.DMA,
        ],
        compiler_params=pltpu.CompilerParams(use_tc_tiling_on_sc=True, needs_layout_passes=False),
    )(_emb_kernel)

    out_tdb = emb(ids_t, table_flat)
    return jnp.transpose(out_tdb, (2, 0, 1))


# speedup vs baseline: 1.0457x; 1.0457x over previous
import functools

import jax
import jax.numpy as jnp
from jax import lax
from jax.experimental import pallas as pl
from jax.experimental.pallas import tpu as pltpu
from jax.experimental.pallas import tpu_sc as plsc

EMBED_DIM = 64
VOCAB_PAD = 1024
NUM_CORES = 2
NUM_SUBCORES = 16
NUM_WORKERS = NUM_CORES * NUM_SUBCORES
T_PER_GROUP = 8
B_BLOCK = 128
LANES = 16


def _emb_kernel(ids_t_hbm, table_hbm, out_hbm,
                table_v, idx0, idx1, tr0, tr1,
                semi0, semi1, semw0, semw1):
    wid = lax.axis_index("s") * NUM_CORES + lax.axis_index("c")
    n_t, n_b = ids_t_hbm.shape
    n_tg = n_t // T_PER_GROUP
    n_bb = n_b // B_BLOCK
    groups = (n_tg * n_bb) // NUM_WORKERS
    g_base = wid * groups

    pltpu.sync_copy(table_hbm, table_v)

    def ids_start(k, idxv, sem):
        g_lin = g_base + k
        tg = g_lin // n_bb
        bb = g_lin % n_bb
        r0 = pl.multiple_of(tg * T_PER_GROUP, T_PER_GROUP)
        c0 = pl.multiple_of(bb * B_BLOCK, B_BLOCK)
        pltpu.async_copy(
            ids_t_hbm.at[pl.ds(r0, T_PER_GROUP), pl.ds(c0, B_BLOCK)],
            idxv, sem)

    def ids_wait(idxv, sem):
        pltpu.make_async_copy(
            ids_t_hbm.at[pl.ds(0, T_PER_GROUP), pl.ds(0, B_BLOCK)],
            idxv, sem).wait()

    def wait_write(tr, sem):
        pltpu.make_async_copy(tr, out_hbm.at[0, :, pl.ds(0, B_BLOCK)],
                              sem).wait()

    trs = (tr0, tr1)
    semws = (semw0, semw1)

    def do_group(k, idxv):
        g_lin = g_base + k
        tg = g_lin // n_bb
        bb = g_lin % n_bb
        c0 = pl.multiple_of(bb * B_BLOCK, B_BLOCK)
        for tt in range(T_PER_GROUP):
            tr = trs[tt % 2]
            semw = semws[tt % 2]

            @pl.when(k * T_PER_GROUP + tt >= 2)
            def _():
                wait_write(tr, semw)

            ids_g = [idxv[tt, pl.ds(g * LANES, LANES)]
                     for g in range(B_BLOCK // LANES)]

            @plsc.parallel_loop(0, EMBED_DIM, unroll=4)
            def dbody(d):
                base = d * VOCAB_PAD
                for g in range(B_BLOCK // LANES):
                    v = plsc.load_gather(table_v, [ids_g[g] + base])
                    tr[d, pl.ds(g * LANES, LANES)] = v
            t = tg * T_PER_GROUP + tt
            pltpu.async_copy(tr, out_hbm.at[t, :, pl.ds(c0, B_BLOCK)], semw)

    ids_start(0, idx0, semi0)
    ids_start(1, idx1, semi1)

    def outer(r, carry):
        k0 = 2 * r
        ids_wait(idx0, semi0)
        do_group(k0, idx0)

        @pl.when(k0 + 2 < groups)
        def _():
            ids_start(k0 + 2, idx0, semi0)
        ids_wait(idx1, semi1)
        do_group(k0 + 1, idx1)

        @pl.when(k0 + 3 < groups)
        def _():
            ids_start(k0 + 3, idx1, semi1)
        return carry

    lax.fori_loop(0, groups // 2, outer, 0)
    wait_write(tr0, semw0)
    wait_write(tr1, semw1)


def kernel(phoneme_ids, table):
    b, t = phoneme_ids.shape
    ids_t = jnp.transpose(phoneme_ids.astype(jnp.int32))
    table_t = jnp.zeros((EMBED_DIM, VOCAB_PAD), jnp.float32)
    table_t = table_t.at[:, :table.shape[0]].set(jnp.transpose(table))
    table_flat = table_t.reshape(-1)

    emb = functools.partial(
        pl.kernel,
        mesh=plsc.VectorSubcoreMesh(core_axis_name="c", subcore_axis_name="s"),
        out_type=jax.ShapeDtypeStruct((t, EMBED_DIM, b), jnp.float32),
        scratch_types=[
            pltpu.VMEM((EMBED_DIM * VOCAB_PAD,), jnp.float32),
            pltpu.VMEM((T_PER_GROUP, B_BLOCK), jnp.int32),
            pltpu.VMEM((T_PER_GROUP, B_BLOCK), jnp.int32),
            pltpu.VMEM((EMBED_DIM, B_BLOCK), jnp.float32),
            pltpu.VMEM((EMBED_DIM, B_BLOCK), jnp.float32),
            pltpu.SemaphoreType.DMA,
            pltpu.SemaphoreType.DMA,
            pltpu.SemaphoreType.DMA,
            pltpu.SemaphoreType.DMA,
        ],
        compiler_params=pltpu.CompilerParams(use_tc_tiling_on_sc=True, needs_layout_passes=False),
    )(_emb_kernel)

    out_tdb = emb(ids_t, table_flat)
    return jnp.transpose(out_tdb, (2, 0, 1))
